# baseline (speedup 1.0000x reference)
import functools

import jax
import jax.numpy as jnp
from jax import lax
from jax.experimental import pallas as pl
from jax.experimental.pallas import tpu as pltpu
from jax.experimental.pallas import tpu_sc as plsc

BATCH = 16384
HIST = 50
D = 64
NC, NS = 2, 16
NW = NC * NS
C = 128
NB_TOT = HIST * (BATCH // C)
NBLK = NB_TOT // NW
CT = BATCH // C

_mesh = plsc.VectorSubcoreMesh(core_axis_name="c", subcore_axis_name="s")


@functools.partial(
    pl.kernel,
    mesh=_mesh,
    out_type=jax.ShapeDtypeStruct((HIST, 8, CT, 8, C), jnp.float32),
    scratch_types=[
        pltpu.VMEM((NBLK, C), jnp.int32),
        pltpu.VMEM((C, D), jnp.float32),
        pltpu.VMEM((C, D), jnp.float32),
        pltpu.VMEM((8, 1, 8, C), jnp.float32),
        pltpu.VMEM((8, 1, 8, C), jnp.float32),
        pltpu.SemaphoreType.DMA,
        pltpu.SemaphoreType.DMA,
        pltpu.SemaphoreType.DMA,
        pltpu.SemaphoreType.DMA,
    ],
    compiler_params=pltpu.CompilerParams(
        use_tc_tiling_on_sc=False, needs_layout_passes=False),
)
def _emb_kernel(idx_hbm, table_hbm, out_hbm, idx_v, g0, g1, t0, t1,
                sg0, sg1, sw0, sw1):
    wid = lax.axis_index("s") * NC + lax.axis_index("c")
    base = wid * NBLK
    pltpu.sync_copy(idx_hbm.at[pl.ds(base, NBLK)], idx_v)

    iota = lax.iota(jnp.int32, 16)

    def fire_gather(i, g, sem):
        pltpu.async_copy(table_hbm.at[idx_v.at[i]], g, sem)

    def drain_gather(g, sem):
        pltpu.make_async_copy(table_hbm.at[idx_v.at[0]], g, sem).wait()

    def transpose_block(g, t):
        for r in range(8):
            for s in range(8):
                dvec = jnp.full((16,), 8 * r + s, jnp.int32)
                for b0 in range(8):
                    vec = plsc.load_gather(g, [iota + 16 * b0, dvec])
                    t[r, 0, s, pl.ds(16 * b0, 16)] = vec

    def fire_write(i, t, sem):
        blk = base + i
        h = blk // CT
        c = blk - h * CT
        pltpu.async_copy(t, out_hbm.at[h, :, pl.ds(c, 1)], sem)

    def drain_write(t, sem):
        pltpu.make_async_copy(t, out_hbm.at[0, :, pl.ds(0, 1)], sem).wait()

    fire_gather(0, g0, sg0)

    def body(tt, carry):
        i = 2 * tt
        fire_gather(i + 1, g1, sg1)
        drain_gather(g0, sg0)

        @pl.when(i > 0)
        def _():
            drain_write(t0, sw0)

        transpose_block(g0, t0)
        fire_write(i, t0, sw0)

        @pl.when(i + 2 < NBLK)
        def _():
            fire_gather(i + 2, g0, sg0)

        drain_gather(g1, sg1)

        @pl.when(i > 0)
        def _():
            drain_write(t1, sw1)

        transpose_block(g1, t1)
        fire_write(i + 1, t1, sw1)
        return carry

    lax.fori_loop(0, NBLK // 2, body, 0)
    drain_write(t0, sw0)
    drain_write(t1, sw1)


def kernel(x, table):
    idx = x.transpose().reshape(NB_TOT, C).astype(jnp.int32)
    o5 = _emb_kernel(idx, table)
    return o5.transpose(2, 4, 0, 1, 3).reshape(BATCH, HIST, D)


# speedup vs baseline: 1.5417x; 1.5417x over previous
import functools

import jax
import jax.numpy as jnp
from jax import lax
from jax.experimental import pallas as pl
from jax.experimental.pallas import tpu as pltpu
from jax.experimental.pallas import tpu_sc as plsc

BATCH = 16384
HIST = 50
D = 64
NC, NS = 2, 16
NW = NC * NS
C = 128
NB_TOT = HIST * (BATCH // C)
NBLK = NB_TOT // NW
CT = BATCH // C

_mesh = plsc.VectorSubcoreMesh(core_axis_name="c", subcore_axis_name="s")


@functools.partial(
    pl.kernel,
    mesh=_mesh,
    out_type=jax.ShapeDtypeStruct((HIST, 8, CT, 8, C), jnp.float32),
    scratch_types=[
        pltpu.VMEM((NBLK, C), jnp.int32),
        pltpu.VMEM((C, D), jnp.float32),
        pltpu.VMEM((C, D), jnp.float32),
        pltpu.VMEM((8, 1, 8, C), jnp.float32),
        pltpu.VMEM((8, 1, 8, C), jnp.float32),
        pltpu.SemaphoreType.DMA,
        pltpu.SemaphoreType.DMA,
        pltpu.SemaphoreType.DMA,
        pltpu.SemaphoreType.DMA,
    ],
    compiler_params=pltpu.CompilerParams(
        use_tc_tiling_on_sc=False, needs_layout_passes=False),
)
def _emb_kernel(idx_hbm, table_hbm, out_hbm, idx_v, g0, g1, t0, t1,
                sg0, sg1, sw0, sw1):
    wid = lax.axis_index("s") * NC + lax.axis_index("c")
    base = wid * NBLK
    pltpu.sync_copy(idx_hbm.at[pl.ds(base, NBLK)], idx_v)

    iota = lax.iota(jnp.int32, 16)

    def fire_gather(i, g, sem):
        pltpu.async_copy(table_hbm.at[idx_v.at[i]], g, sem)

    def drain_gather(g, sem):
        pltpu.make_async_copy(table_hbm.at[idx_v.at[0]], g, sem).wait()

    zero16 = jnp.zeros((16,), jnp.int32)

    def transpose_block(g, t):
        @plsc.parallel_loop(0, C, step=1, unroll=8)
        def _(b):
            bs = zero16 + b
            for k in range(D // 16):
                vec = g[b, pl.ds(16 * k, 16)]
                dv = iota + 16 * k
                plsc.store_scatter(
                    t,
                    [lax.shift_right_logical(dv, 3), zero16,
                     lax.bitwise_and(dv, 7), bs],
                    vec)

    def fire_write(i, t, sem):
        blk = base + i
        h = blk // CT
        c = blk - h * CT
        pltpu.async_copy(t, out_hbm.at[h, :, pl.ds(c, 1)], sem)

    def drain_write(t, sem):
        pltpu.make_async_copy(t, out_hbm.at[0, :, pl.ds(0, 1)], sem).wait()

    fire_gather(0, g0, sg0)

    def body(tt, carry):
        i = 2 * tt
        fire_gather(i + 1, g1, sg1)
        drain_gather(g0, sg0)

        @pl.when(i > 0)
        def _():
            drain_write(t0, sw0)

        transpose_block(g0, t0)
        fire_write(i, t0, sw0)

        @pl.when(i + 2 < NBLK)
        def _():
            fire_gather(i + 2, g0, sg0)

        drain_gather(g1, sg1)

        @pl.when(i > 0)
        def _():
            drain_write(t1, sw1)

        transpose_block(g1, t1)
        fire_write(i + 1, t1, sw1)
        return carry

    lax.fori_loop(0, NBLK // 2, body, 0)
    drain_write(t0, sw0)
    drain_write(t1, sw1)


def kernel(x, table):
    idx = x.transpose().reshape(NB_TOT, C).astype(jnp.int32)
    o5 = _emb_kernel(idx, table)
    return o5.transpose(2, 4, 0, 1, 3).reshape(BATCH, HIST, D)


# speedup vs baseline: 2.1453x; 1.3915x over previous
import functools

import jax
import jax.numpy as jnp
from jax import lax
from jax.experimental import pallas as pl
from jax.experimental.pallas import tpu as pltpu
from jax.experimental.pallas import tpu_sc as plsc

BATCH = 16384
HIST = 50
D = 64
NC, NS = 2, 16
NW = NC * NS
C = 128
HB = BATCH // 2
NB_TOT = HIST * (BATCH // C)
NBLK = NB_TOT // NW
CT = BATCH // C

_mesh = plsc.VectorSubcoreMesh(core_axis_name="c", subcore_axis_name="s")


@functools.partial(
    pl.kernel,
    mesh=_mesh,
    out_type=jax.ShapeDtypeStruct((HIST * HB, 2 * D), jnp.float32),
    scratch_types=[
        pltpu.VMEM((NBLK, C), jnp.int32),
        pltpu.VMEM((C, D), jnp.float32),
        pltpu.VMEM((C, D), jnp.float32),
        pltpu.SemaphoreType.DMA,
        pltpu.SemaphoreType.DMA,
        pltpu.SemaphoreType.DMA,
        pltpu.SemaphoreType.DMA,
    ],
    compiler_params=pltpu.CompilerParams(use_tc_tiling_on_sc=False),
)
def _sc_gather(idx_hbm, table_hbm, out_hbm, idx_v, g0, g1, sg0, sg1, sw0, sw1):
    wid = lax.axis_index("s") * NC + lax.axis_index("c")
    base = wid * NBLK
    pltpu.sync_copy(idx_hbm.at[pl.ds(base, NBLK)], idx_v)

    def fire_gather(i, g, sem):
        pltpu.async_copy(table_hbm.at[idx_v.at[i]], g, sem)

    def drain_gather(g, sem):
        pltpu.make_async_copy(table_hbm.at[idx_v.at[0]], g, sem).wait()

    def fire_write(i, g, sem):
        blk = base + i
        h = blk // CT
        c = blk - h * CT
        q = c // (CT // 2)
        c2 = c - q * (CT // 2)
        pltpu.async_copy(
            g, out_hbm.at[pl.ds(h * HB + c2 * C, C), pl.ds(q * D, D)], sem)

    def drain_write(g, sem):
        pltpu.make_async_copy(
            g, out_hbm.at[pl.ds(0, C), pl.ds(0, D)], sem).wait()

    fire_gather(0, g0, sg0)

    def body(t, carry):
        i = 2 * t
        fire_gather(i + 1, g1, sg1)
        drain_gather(g0, sg0)

        @pl.when(i > 0)
        def _():
            drain_write(g0, sw0)

        fire_write(i, g0, sw0)

        @pl.when(i + 2 < NBLK)
        def _():
            fire_gather(i + 2, g0, sg0)

        drain_gather(g1, sg1)

        @pl.when(i > 0)
        def _():
            drain_write(g1, sw1)

        fire_write(i + 1, g1, sw1)
        return carry

    lax.fori_loop(0, NBLK // 2, body, 0)
    drain_write(g0, sw0)
    drain_write(g1, sw1)


def _tc_body(in_ref, out_ref):
    v = in_ref[...]
    out_ref[0, :, 0:HB] = v[:, 0:D].T
    out_ref[0, :, HB:BATCH] = v[:, D:2 * D].T


_tc_transpose = pl.pallas_call(
    _tc_body,
    grid=(HIST,),
    in_specs=[pl.BlockSpec((HB, 2 * D), lambda h: (h, 0))],
    out_specs=pl.BlockSpec((1, D, BATCH), lambda h: (h, 0, 0)),
    out_shape=jax.ShapeDtypeStruct((HIST, D, BATCH), jnp.float32),
)


def kernel(x, table):
    idx = x.transpose().reshape(NB_TOT, C).astype(jnp.int32)
    inter = _sc_gather(idx, table)
    o3 = _tc_transpose(inter)
    return o3.transpose(2, 0, 1)


# speedup vs baseline: 2.2506x; 1.0491x over previous
import functools

import jax
import jax.numpy as jnp
from jax import lax
from jax.experimental import pallas as pl
from jax.experimental.pallas import tpu as pltpu
from jax.experimental.pallas import tpu_sc as plsc

BATCH = 16384
HIST = 50
D = 64
NC, NS = 2, 16
NW = NC * NS
C = 128
HB = BATCH // 2
NB_TOT = HIST * (BATCH // C)
NBLK = NB_TOT // NW
CT = BATCH // C

_mesh = plsc.VectorSubcoreMesh(core_axis_name="c", subcore_axis_name="s")


@functools.partial(
    pl.kernel,
    mesh=_mesh,
    out_type=jax.ShapeDtypeStruct((HIST * HB, 2 * D), jnp.float32),
    scratch_types=[
        pltpu.VMEM((NBLK, C), jnp.int32),
        pltpu.VMEM((C, D), jnp.float32),
        pltpu.VMEM((C, D), jnp.float32),
        pltpu.SemaphoreType.DMA,
        pltpu.SemaphoreType.DMA,
        pltpu.SemaphoreType.DMA,
        pltpu.SemaphoreType.DMA,
    ],
    compiler_params=pltpu.CompilerParams(use_tc_tiling_on_sc=False),
)
def _sc_gather(idx_hbm, table_hbm, out_hbm, idx_v, g0, g1, sg0, sg1, sw0, sw1):
    wid = lax.axis_index("s") * NC + lax.axis_index("c")
    base = wid * NBLK
    pltpu.sync_copy(idx_hbm.at[pl.ds(base, NBLK)], idx_v)

    def fire_gather(i, g, sem):
        pltpu.async_copy(table_hbm.at[idx_v.at[i]], g, sem)

    def drain_gather(g, sem):
        pltpu.make_async_copy(table_hbm.at[idx_v.at[0]], g, sem).wait()

    def fire_write(i, g, sem):
        blk = base + i
        h = blk // CT
        c = blk - h * CT
        q = c // (CT // 2)
        c2 = c - q * (CT // 2)
        pltpu.async_copy(
            g, out_hbm.at[pl.ds(h * HB + c2 * C, C), pl.ds(q * D, D)], sem)

    def drain_write(g, sem):
        pltpu.make_async_copy(
            g, out_hbm.at[pl.ds(0, C), pl.ds(0, D)], sem).wait()

    fire_gather(0, g0, sg0)

    def body(t, carry):
        i = 2 * t
        fire_gather(i + 1, g1, sg1)
        drain_gather(g0, sg0)

        @pl.when(i > 0)
        def _():
            drain_write(g0, sw0)

        fire_write(i, g0, sw0)

        @pl.when(i + 2 < NBLK)
        def _():
            fire_gather(i + 2, g0, sg0)

        drain_gather(g1, sg1)

        @pl.when(i > 0)
        def _():
            drain_write(g1, sw1)

        fire_write(i + 1, g1, sw1)
        return carry

    lax.fori_loop(0, NBLK // 2, body, 0)
    drain_write(g0, sw0)
    drain_write(g1, sw1)


def _tc_body(in_ref, out_ref):
    w = in_ref[...].T
    out_ref[0, :, 0:HB] = w[0:D, :]
    out_ref[0, :, HB:BATCH] = w[D:2 * D, :]


_tc_transpose = pl.pallas_call(
    _tc_body,
    grid=(HIST,),
    in_specs=[pl.BlockSpec((HB, 2 * D), lambda h: (h, 0))],
    out_specs=pl.BlockSpec((1, D, BATCH), lambda h: (h, 0, 0)),
    out_shape=jax.ShapeDtypeStruct((HIST, D, BATCH), jnp.float32),
)


def kernel(x, table):
    idx = x.transpose().reshape(NB_TOT, C).astype(jnp.int32)
    inter = _sc_gather(idx, table)
    o3 = _tc_transpose(inter)
    return o3.transpose(2, 0, 1)
